# Initial kernel scaffold; baseline (speedup 1.0000x reference)
import jax
import jax.numpy as jnp
from jax.experimental import pallas as pl


def kernel(mol_vec, parent_edge_index, sibling_edge_index, node_level, W_parent, b_parent, W_cls, b_cls):
    raise NotImplementedError("write your pallas kernel here")


# speedup vs baseline: 2.3202x; 2.3202x over previous
import functools

import jax
import jax.numpy as jnp
from jax import lax
from jax.experimental import pallas as pl
from jax.experimental.pallas import tpu as pltpu
from jax.experimental.pallas import tpu_sc as plsc

_N = 10000
_D = 128
_NUM_LEVELS = 8

_NC, _NS = 2, 16
_NW = _NC * _NS
_CHUNK = 128
_NPAD = 10240
_RPS = _NPAD // _NS
_ZCH = _RPS // _CHUNK

_BLK = 1000
_GRID = _N // _BLK


@functools.lru_cache(maxsize=None)
def _make_seg_sum(epad: int, width: int):
  ept = epad // _NW
  nchunks = ept // _CHUNK
  mesh = plsc.VectorSubcoreMesh(core_axis_name="c", subcore_axis_name="s")
  out_type = jax.ShapeDtypeStruct((_NC, _NPAD, width), jnp.float32)
  scratch = [
      pltpu.VMEM((_CHUNK,), jnp.int32),
      pltpu.VMEM((_CHUNK,), jnp.int32),
      pltpu.VMEM((_CHUNK, width), jnp.float32),
      pltpu.VMEM_SHARED((_NPAD, width), jnp.float32),
  ]

  def body(data, src, dst, zeros, out, src_v, dst_v, rows_v, acc):
    c = lax.axis_index("c")
    s = lax.axis_index("s")
    base = s * _RPS

    pltpu.sync_copy(zeros, rows_v)

    @pl.loop(0, _ZCH)
    def _(k):
      pltpu.sync_copy(rows_v, acc.at[pl.ds(base + k * _CHUNK, _CHUNK), :])

    plsc.subcore_barrier()

    wid = c * _NS + s

    @pl.loop(0, nchunks)
    def _(ch):
      off = wid * ept + ch * _CHUNK
      pltpu.sync_copy(src.at[pl.ds(off, _CHUNK)], src_v)
      pltpu.sync_copy(dst.at[pl.ds(off, _CHUNK)], dst_v)
      pltpu.sync_copy(data.at[src_v], rows_v)
      pltpu.sync_copy(rows_v, acc.at[dst_v], add=True)

    plsc.subcore_barrier()

    @pl.loop(0, _ZCH)
    def _(k):
      r = base + k * _CHUNK
      pltpu.sync_copy(acc.at[pl.ds(r, _CHUNK), :], rows_v)
      pltpu.sync_copy(rows_v, out.at[c, pl.ds(r, _CHUNK), :])

  return pl.kernel(body, out_type=out_type, mesh=mesh, scratch_types=scratch)


def _prep_edges(edge_index):
  e = edge_index.shape[1]
  epad = -(-e // (_NW * _CHUNK)) * (_NW * _CHUNK)
  src = jnp.concatenate(
      [edge_index[0].astype(jnp.int32), jnp.zeros((epad - e,), jnp.int32)])
  dst = jnp.concatenate(
      [edge_index[1].astype(jnp.int32), jnp.full((epad - e,), _N, jnp.int32)])
  return src, dst, epad


def _sibdiv_call(ssum, cnt):
  def body(ss_ref, cnt_ref, o_ref):
    total = ss_ref[0] + ss_ref[1]
    c = cnt_ref[0, :, 0:1] + cnt_ref[1, :, 0:1]
    o_ref[...] = total / jnp.maximum(c, 1.0)

  return pl.pallas_call(
      body,
      grid=(_GRID,),
      in_specs=[pl.BlockSpec((_NC, _BLK, _D), lambda i: (0, i, 0)),
                pl.BlockSpec((_NC, _BLK, _D), lambda i: (0, i, 0))],
      out_specs=pl.BlockSpec((_BLK, _D), lambda i: (i, 0)),
      out_shape=jax.ShapeDtypeStruct((_N, _D), jnp.float32),
  )(ssum, cnt)


def _level1_call(sib_h, lev2d, b2d):
  def body(sib_ref, lev_ref, b_ref, o_ref):
    upd = jnp.maximum(b_ref[...], 0.0) + sib_ref[...]
    o_ref[...] = jnp.where(lev_ref[...] == 1, upd, 0.0)

  return pl.pallas_call(
      body,
      grid=(_GRID,),
      in_specs=[pl.BlockSpec((_BLK, _D), lambda i: (i, 0)),
                pl.BlockSpec((_BLK, 1), lambda i: (i, 0)),
                pl.BlockSpec((1, _D), lambda i: (0, 0))],
      out_specs=pl.BlockSpec((_BLK, _D), lambda i: (i, 0)),
      out_shape=jax.ShapeDtypeStruct((_N, _D), jnp.float32),
  )(sib_h, lev2d, b2d)


def _update_call(lvl, z, pz, sib_h, lev2d, W, b2d):
  def body(z_ref, pz_ref, sib_ref, lev_ref, w_ref, b_ref, o_ref):
    acc = pz_ref[0] + pz_ref[1]
    h = jnp.dot(acc, w_ref[...], preferred_element_type=jnp.float32) + b_ref[...]
    upd = jnp.maximum(h, 0.0) + sib_ref[...]
    o_ref[...] = z_ref[...] + jnp.where(lev_ref[...] == lvl, upd, 0.0)

  return pl.pallas_call(
      body,
      grid=(_GRID,),
      in_specs=[pl.BlockSpec((_BLK, _D), lambda i: (i, 0)),
                pl.BlockSpec((_NC, _BLK, _D), lambda i: (0, i, 0)),
                pl.BlockSpec((_BLK, _D), lambda i: (i, 0)),
                pl.BlockSpec((_BLK, 1), lambda i: (i, 0)),
                pl.BlockSpec((_D, _D), lambda i: (0, 0)),
                pl.BlockSpec((1, _D), lambda i: (0, 0))],
      out_specs=pl.BlockSpec((_BLK, _D), lambda i: (i, 0)),
      out_shape=jax.ShapeDtypeStruct((_N, _D), jnp.float32),
  )(z, pz, sib_h, lev2d, W, b2d)


def _logits_call(z, Wp, bp, cpad):
  def body(z_ref, w_ref, b_ref, o_ref):
    o_ref[...] = jnp.dot(z_ref[...], w_ref[...],
                         preferred_element_type=jnp.float32) + b_ref[...]

  return pl.pallas_call(
      body,
      grid=(_GRID,),
      in_specs=[pl.BlockSpec((_BLK, _D), lambda i: (i, 0)),
                pl.BlockSpec((_D, cpad), lambda i: (0, 0)),
                pl.BlockSpec((1, cpad), lambda i: (0, 0))],
      out_specs=pl.BlockSpec((_BLK, cpad), lambda i: (i, 0)),
      out_shape=jax.ShapeDtypeStruct((_N, cpad), jnp.float32),
  )(z, Wp, bp)


def kernel(mol_vec, parent_edge_index, sibling_edge_index, node_level,
           W_parent, b_parent, W_cls, b_cls):
  zeros128 = jnp.zeros((_CHUNK, _D), jnp.float32)
  ones_tab = jnp.ones((_N, _D), jnp.float32)
  s_src, s_dst, s_epad = _prep_edges(sibling_edge_index)
  seg_kern = _make_seg_sum(s_epad, _D)
  sib_sum = seg_kern(mol_vec, s_src, s_dst, zeros128)
  sib_cnt = seg_kern(ones_tab, s_src, s_dst, zeros128)
  sib_h = _sibdiv_call(sib_sum, sib_cnt)

  lev2d = node_level.reshape(_N, 1).astype(jnp.int32)
  b2d = b_parent.reshape(1, _D)

  z = _level1_call(sib_h, lev2d, b2d)

  p_src, p_dst, p_epad = _prep_edges(parent_edge_index)
  pseg_kern = _make_seg_sum(p_epad, _D)
  for lvl in range(2, _NUM_LEVELS):
    pz = pseg_kern(z, p_src, p_dst, zeros128)
    z = _update_call(lvl, z, pz, sib_h, lev2d, W_parent, b2d)

  ncls = W_cls.shape[1]
  cpad = -(-ncls // 128) * 128
  Wp = jnp.pad(W_cls, ((0, 0), (0, cpad - ncls)))
  bp = jnp.pad(b_cls, (0, cpad - ncls)).reshape(1, cpad)
  logits = _logits_call(z, Wp, bp, cpad)
  return logits[:, :ncls]


# speedup vs baseline: 4.2409x; 1.8278x over previous
import functools

import jax
import jax.numpy as jnp
from jax import lax
from jax.experimental import pallas as pl
from jax.experimental.pallas import tpu as pltpu
from jax.experimental.pallas import tpu_sc as plsc

_N = 10000
_D = 128
_NUM_LEVELS = 8
_E = 160000

_NC, _NS = 2, 16
_NW = _NC * _NS
_CHUNK = 128
_NCHUNKS = _E // _CHUNK
_CPW = _NCHUNKS // _NW
_XTRA = _NCHUNKS % _NW
_NPAD = 10240
_RPS = _NPAD // _NS
_ZCH = _RPS // _CHUNK

_BLK = 1000
_GRID = _N // _BLK


def _worker_edge_loop(src, dst, data, acc, src_v, dst_v, rows_v, wid, extra):
  def do_chunk(off):
    pltpu.sync_copy(src.at[pl.ds(off, _CHUNK)], src_v)
    pltpu.sync_copy(dst.at[pl.ds(off, _CHUNK)], dst_v)
    pltpu.sync_copy(data.at[src_v], rows_v)
    pltpu.sync_copy(rows_v, acc.at[dst_v], add=True)
    extra(dst_v)

  @pl.loop(0, _CPW)
  def _(ch):
    do_chunk((wid * _CPW + ch) * _CHUNK)

  @pl.when(wid < _XTRA)
  def _():
    do_chunk((_NW * _CPW + wid) * _CHUNK)


def _make_seg_sum(width: int):
  mesh = plsc.VectorSubcoreMesh(core_axis_name="c", subcore_axis_name="s")
  out_type = jax.ShapeDtypeStruct((_NC, _NPAD, width), jnp.float32)
  scratch = [
      pltpu.VMEM((_CHUNK,), jnp.int32),
      pltpu.VMEM((_CHUNK,), jnp.int32),
      pltpu.VMEM((_CHUNK, width), jnp.float32),
      pltpu.VMEM_SHARED((_NPAD, width), jnp.float32),
  ]

  def body(data, src, dst, zeros, out, src_v, dst_v, rows_v, acc):
    c = lax.axis_index("c")
    s = lax.axis_index("s")
    base = s * _RPS

    pltpu.sync_copy(zeros, rows_v)

    @pl.loop(0, _ZCH)
    def _(k):
      pltpu.sync_copy(rows_v, acc.at[pl.ds(base + k * _CHUNK, _CHUNK), :])

    plsc.subcore_barrier()

    wid = c * _NS + s

    def extra(dref):
      del dref

    _worker_edge_loop(src, dst, data, acc, src_v, dst_v, rows_v, wid, extra)

    plsc.subcore_barrier()

    @pl.loop(0, _ZCH)
    def _(k):
      r = base + k * _CHUNK
      pltpu.sync_copy(acc.at[pl.ds(r, _CHUNK), :], rows_v)
      pltpu.sync_copy(rows_v, out.at[c, pl.ds(r, _CHUNK), :])

  return pl.kernel(body, out_type=out_type, mesh=mesh, scratch_types=scratch)


_seg_sum = functools.lru_cache(maxsize=None)(_make_seg_sum)


def _sibdiv_call(ssum, cnt):
  def body(ss_ref, cnt_ref, o_ref):
    total = ss_ref[0] + ss_ref[1]
    c = cnt_ref[0, :, 0:1] + cnt_ref[1, :, 0:1]
    o_ref[...] = total / jnp.maximum(c, 1.0)

  return pl.pallas_call(
      body,
      grid=(_GRID,),
      in_specs=[pl.BlockSpec((_NC, _BLK, _D), lambda i: (0, i, 0)),
                pl.BlockSpec((_NC, _BLK, _D), lambda i: (0, i, 0))],
      out_specs=pl.BlockSpec((_BLK, _D), lambda i: (i, 0)),
      out_shape=jax.ShapeDtypeStruct((_N, _D), jnp.float32),
  )(ssum, cnt)


def _level1_call(sib_h, lev2d, b2d):
  def body(sib_ref, lev_ref, b_ref, o_ref):
    upd = jnp.maximum(b_ref[...], 0.0) + sib_ref[...]
    o_ref[...] = jnp.where(lev_ref[...] == 1, upd, 0.0)

  return pl.pallas_call(
      body,
      grid=(_GRID,),
      in_specs=[pl.BlockSpec((_BLK, _D), lambda i: (i, 0)),
                pl.BlockSpec((_BLK, 1), lambda i: (i, 0)),
                pl.BlockSpec((1, _D), lambda i: (0, 0))],
      out_specs=pl.BlockSpec((_BLK, _D), lambda i: (i, 0)),
      out_shape=jax.ShapeDtypeStruct((_N, _D), jnp.float32),
  )(sib_h, lev2d, b2d)


def _update_call(lvl, z, pz, sib_h, lev2d, W, b2d):
  def body(z_ref, pz_ref, sib_ref, lev_ref, w_ref, b_ref, o_ref):
    acc = pz_ref[0] + pz_ref[1]
    h = jnp.dot(acc, w_ref[...], preferred_element_type=jnp.float32) + b_ref[...]
    upd = jnp.maximum(h, 0.0) + sib_ref[...]
    o_ref[...] = z_ref[...] + jnp.where(lev_ref[...] == lvl, upd, 0.0)

  return pl.pallas_call(
      body,
      grid=(_GRID,),
      in_specs=[pl.BlockSpec((_BLK, _D), lambda i: (i, 0)),
                pl.BlockSpec((_NC, _BLK, _D), lambda i: (0, i, 0)),
                pl.BlockSpec((_BLK, _D), lambda i: (i, 0)),
                pl.BlockSpec((_BLK, 1), lambda i: (i, 0)),
                pl.BlockSpec((_D, _D), lambda i: (0, 0)),
                pl.BlockSpec((1, _D), lambda i: (0, 0))],
      out_specs=pl.BlockSpec((_BLK, _D), lambda i: (i, 0)),
      out_shape=jax.ShapeDtypeStruct((_N, _D), jnp.float32),
  )(z, pz, sib_h, lev2d, W, b2d)


def _logits_call(z, Wp, bp, cpad):
  def body(z_ref, w_ref, b_ref, o_ref):
    o_ref[...] = jnp.dot(z_ref[...], w_ref[...],
                         preferred_element_type=jnp.float32) + b_ref[...]

  return pl.pallas_call(
      body,
      grid=(_GRID,),
      in_specs=[pl.BlockSpec((_BLK, _D), lambda i: (i, 0)),
                pl.BlockSpec((_D, cpad), lambda i: (0, 0)),
                pl.BlockSpec((1, cpad), lambda i: (0, 0))],
      out_specs=pl.BlockSpec((_BLK, cpad), lambda i: (i, 0)),
      out_shape=jax.ShapeDtypeStruct((_N, cpad), jnp.float32),
  )(z, Wp, bp)


def kernel(mol_vec, parent_edge_index, sibling_edge_index, node_level,
           W_parent, b_parent, W_cls, b_cls):
  zeros128 = jnp.zeros((_CHUNK, _D), jnp.float32)
  ones_tab = jnp.ones((_N, _D), jnp.float32)

  s_src = sibling_edge_index[0].astype(jnp.int32)
  s_dst = sibling_edge_index[1].astype(jnp.int32)
  seg = _seg_sum(_D)
  sib_sum = seg(mol_vec, s_src, s_dst, zeros128)
  sib_cnt = seg(ones_tab, s_src, s_dst, zeros128)
  sib_h = _sibdiv_call(sib_sum, sib_cnt)

  lev2d = node_level.reshape(_N, 1).astype(jnp.int32)
  b2d = b_parent.reshape(1, _D)

  z = _level1_call(sib_h, lev2d, b2d)

  p_src = parent_edge_index[0].astype(jnp.int32)
  p_dst = parent_edge_index[1].astype(jnp.int32)
  for lvl in range(2, _NUM_LEVELS):
    pz = seg(z, p_src, p_dst, zeros128)
    z = _update_call(lvl, z, pz, sib_h, lev2d, W_parent, b2d)

  ncls = W_cls.shape[1]
  cpad = -(-ncls // 128) * 128
  Wp = jnp.pad(W_cls, ((0, 0), (0, cpad - ncls)))
  bp = jnp.pad(b_cls, (0, cpad - ncls)).reshape(1, cpad)
  logits = _logits_call(z, Wp, bp, cpad)
  return logits[:, :ncls]


# speedup vs baseline: 5.8308x; 1.3749x over previous
import functools

import jax
import jax.numpy as jnp
from jax import lax
from jax.experimental import pallas as pl
from jax.experimental.pallas import tpu as pltpu
from jax.experimental.pallas import tpu_sc as plsc

_N = 10000
_D = 128
_NUM_LEVELS = 8
_E = 160000

_NC, _NS = 2, 16
_NW = _NC * _NS
_CHUNK = 128
_NCHUNKS = _E // _CHUNK
_CPW = _NCHUNKS // _NW
_XTRA = _NCHUNKS % _NW
_NPAD = 10240
_RPS = _NPAD // _NS
_ZCH = _RPS // _CHUNK

_BLK = 1000
_GRID = _N // _BLK


def _worker_edge_loop(src, dst, data, acc, bufs, wid):
  (src_a, dst_a, rows_a, sem_a), (src_b, dst_b, rows_b, sem_b) = bufs

  def start(ch, sv, dv, rv, sem):
    off = (wid * _CPW + ch) * _CHUNK
    pltpu.sync_copy(src.at[pl.ds(off, _CHUNK)], sv)
    pltpu.sync_copy(dst.at[pl.ds(off, _CHUNK)], dv)
    pltpu.async_copy(data.at[sv], rv, sem)

  def finish(sv, dv, rv, sem):
    pltpu.make_async_copy(data.at[sv], rv, sem).wait()
    pltpu.sync_copy(rv, acc.at[dv], add=True)

  start(0, src_a, dst_a, rows_a, sem_a)

  @pl.loop(0, (_CPW - 1) // 2)
  def _(h):
    start(2 * h + 1, src_b, dst_b, rows_b, sem_b)
    finish(src_a, dst_a, rows_a, sem_a)
    start(2 * h + 2, src_a, dst_a, rows_a, sem_a)
    finish(src_b, dst_b, rows_b, sem_b)

  finish(src_a, dst_a, rows_a, sem_a)

  @pl.when(wid < _XTRA)
  def _():
    off = (_NW * _CPW + wid) * _CHUNK
    pltpu.sync_copy(src.at[pl.ds(off, _CHUNK)], src_a)
    pltpu.sync_copy(dst.at[pl.ds(off, _CHUNK)], dst_a)
    pltpu.sync_copy(data.at[src_a], rows_a)
    pltpu.sync_copy(rows_a, acc.at[dst_a], add=True)


def _make_seg_sum(width: int):
  mesh = plsc.VectorSubcoreMesh(core_axis_name="c", subcore_axis_name="s")
  out_type = jax.ShapeDtypeStruct((_NC, _NPAD, width), jnp.float32)
  scratch = [
      pltpu.VMEM((_CHUNK,), jnp.int32),
      pltpu.VMEM((_CHUNK,), jnp.int32),
      pltpu.VMEM((_CHUNK, width), jnp.float32),
      pltpu.SemaphoreType.DMA,
      pltpu.VMEM((_CHUNK,), jnp.int32),
      pltpu.VMEM((_CHUNK,), jnp.int32),
      pltpu.VMEM((_CHUNK, width), jnp.float32),
      pltpu.SemaphoreType.DMA,
      pltpu.VMEM_SHARED((_NPAD, width), jnp.float32),
  ]

  def body(data, src, dst, zeros, out,
           src_a, dst_a, rows_a, sem_a, src_b, dst_b, rows_b, sem_b, acc):
    c = lax.axis_index("c")
    s = lax.axis_index("s")
    base = s * _RPS

    pltpu.sync_copy(zeros, rows_a)

    @pl.loop(0, _ZCH)
    def _(k):
      pltpu.sync_copy(rows_a, acc.at[pl.ds(base + k * _CHUNK, _CHUNK), :])

    plsc.subcore_barrier()

    wid = c * _NS + s
    bufs = ((src_a, dst_a, rows_a, sem_a), (src_b, dst_b, rows_b, sem_b))
    _worker_edge_loop(src, dst, data, acc, bufs, wid)

    plsc.subcore_barrier()

    @pl.loop(0, _ZCH)
    def _(k):
      r = base + k * _CHUNK
      pltpu.sync_copy(acc.at[pl.ds(r, _CHUNK), :], rows_a)
      pltpu.sync_copy(rows_a, out.at[c, pl.ds(r, _CHUNK), :])

  return pl.kernel(body, out_type=out_type, mesh=mesh, scratch_types=scratch)


_seg_sum = functools.lru_cache(maxsize=None)(_make_seg_sum)


def _sibdiv_call(ssum, cnt):
  def body(ss_ref, cnt_ref, o_ref):
    total = ss_ref[0] + ss_ref[1]
    c = cnt_ref[0, :, 0:1] + cnt_ref[1, :, 0:1]
    o_ref[...] = total / jnp.maximum(c, 1.0)

  return pl.pallas_call(
      body,
      grid=(_GRID,),
      in_specs=[pl.BlockSpec((_NC, _BLK, _D), lambda i: (0, i, 0)),
                pl.BlockSpec((_NC, _BLK, _D), lambda i: (0, i, 0))],
      out_specs=pl.BlockSpec((_BLK, _D), lambda i: (i, 0)),
      out_shape=jax.ShapeDtypeStruct((_N, _D), jnp.float32),
  )(ssum, cnt)


def _level1_call(sib_h, lev2d, b2d):
  def body(sib_ref, lev_ref, b_ref, o_ref):
    upd = jnp.maximum(b_ref[...], 0.0) + sib_ref[...]
    o_ref[...] = jnp.where(lev_ref[...] == 1, upd, 0.0)

  return pl.pallas_call(
      body,
      grid=(_GRID,),
      in_specs=[pl.BlockSpec((_BLK, _D), lambda i: (i, 0)),
                pl.BlockSpec((_BLK, 1), lambda i: (i, 0)),
                pl.BlockSpec((1, _D), lambda i: (0, 0))],
      out_specs=pl.BlockSpec((_BLK, _D), lambda i: (i, 0)),
      out_shape=jax.ShapeDtypeStruct((_N, _D), jnp.float32),
  )(sib_h, lev2d, b2d)


def _update_call(lvl, z, pz, sib_h, lev2d, W, b2d):
  def body(z_ref, pz_ref, sib_ref, lev_ref, w_ref, b_ref, o_ref):
    acc = pz_ref[0] + pz_ref[1]
    h = jnp.dot(acc, w_ref[...], preferred_element_type=jnp.float32) + b_ref[...]
    upd = jnp.maximum(h, 0.0) + sib_ref[...]
    o_ref[...] = z_ref[...] + jnp.where(lev_ref[...] == lvl, upd, 0.0)

  return pl.pallas_call(
      body,
      grid=(_GRID,),
      in_specs=[pl.BlockSpec((_BLK, _D), lambda i: (i, 0)),
                pl.BlockSpec((_NC, _BLK, _D), lambda i: (0, i, 0)),
                pl.BlockSpec((_BLK, _D), lambda i: (i, 0)),
                pl.BlockSpec((_BLK, 1), lambda i: (i, 0)),
                pl.BlockSpec((_D, _D), lambda i: (0, 0)),
                pl.BlockSpec((1, _D), lambda i: (0, 0))],
      out_specs=pl.BlockSpec((_BLK, _D), lambda i: (i, 0)),
      out_shape=jax.ShapeDtypeStruct((_N, _D), jnp.float32),
  )(z, pz, sib_h, lev2d, W, b2d)


def _logits_call(z, Wp, bp, cpad):
  def body(z_ref, w_ref, b_ref, o_ref):
    o_ref[...] = jnp.dot(z_ref[...], w_ref[...],
                         preferred_element_type=jnp.float32) + b_ref[...]

  return pl.pallas_call(
      body,
      grid=(_GRID,),
      in_specs=[pl.BlockSpec((_BLK, _D), lambda i: (i, 0)),
                pl.BlockSpec((_D, cpad), lambda i: (0, 0)),
                pl.BlockSpec((1, cpad), lambda i: (0, 0))],
      out_specs=pl.BlockSpec((_BLK, cpad), lambda i: (i, 0)),
      out_shape=jax.ShapeDtypeStruct((_N, cpad), jnp.float32),
  )(z, Wp, bp)


def kernel(mol_vec, parent_edge_index, sibling_edge_index, node_level,
           W_parent, b_parent, W_cls, b_cls):
  zeros128 = jnp.zeros((_CHUNK, _D), jnp.float32)
  ones_tab = jnp.ones((_N, _D), jnp.float32)

  s_src = sibling_edge_index[0].astype(jnp.int32)
  s_dst = sibling_edge_index[1].astype(jnp.int32)
  seg = _seg_sum(_D)
  sib_sum = seg(mol_vec, s_src, s_dst, zeros128)
  sib_cnt = seg(ones_tab, s_src, s_dst, zeros128)
  sib_h = _sibdiv_call(sib_sum, sib_cnt)

  lev2d = node_level.reshape(_N, 1).astype(jnp.int32)
  b2d = b_parent.reshape(1, _D)

  z = _level1_call(sib_h, lev2d, b2d)

  p_src = parent_edge_index[0].astype(jnp.int32)
  p_dst = parent_edge_index[1].astype(jnp.int32)
  for lvl in range(2, _NUM_LEVELS):
    pz = seg(z, p_src, p_dst, zeros128)
    z = _update_call(lvl, z, pz, sib_h, lev2d, W_parent, b2d)

  ncls = W_cls.shape[1]
  cpad = -(-ncls // 128) * 128
  Wp = jnp.pad(W_cls, ((0, 0), (0, cpad - ncls)))
  bp = jnp.pad(b_cls, (0, cpad - ncls)).reshape(1, cpad)
  logits = _logits_call(z, Wp, bp, cpad)
  return logits[:, :ncls]


# speedup vs baseline: 5.9318x; 1.0173x over previous
import functools

import jax
import jax.numpy as jnp
from jax import lax
from jax.experimental import pallas as pl
from jax.experimental.pallas import tpu as pltpu
from jax.experimental.pallas import tpu_sc as plsc

_N = 10000
_D = 128
_NUM_LEVELS = 8
_E = 160000

_NC, _NS = 2, 16
_NW = _NC * _NS
_CHUNK = 128
_NCHUNKS = _E // _CHUNK
_CPW = _NCHUNKS // _NW
_XTRA = _NCHUNKS % _NW
_NPAD = 10240
_RPS = _NPAD // _NS
_ZCH = _RPS // _CHUNK

_BLK = 1000
_GRID = _N // _BLK


def _worker_edge_loop(edges, data, acc, bufs, wid):
  (src_a, dst_a, rows_a, sem_a), (src_b, dst_b, rows_b, sem_b) = bufs

  def start(ch, sv, dv, rv, sem):
    off = (wid * _CPW + ch) * _CHUNK
    pltpu.sync_copy(edges.at[pl.ds(off, _CHUNK)], sv)
    pltpu.sync_copy(edges.at[pl.ds(_E + off, _CHUNK)], dv)
    pltpu.async_copy(data.at[sv], rv, sem)

  def finish(sv, dv, rv, sem):
    pltpu.make_async_copy(data.at[sv], rv, sem).wait()
    pltpu.sync_copy(rv, acc.at[dv], add=True)

  start(0, src_a, dst_a, rows_a, sem_a)

  @pl.loop(0, (_CPW - 1) // 2)
  def _(h):
    start(2 * h + 1, src_b, dst_b, rows_b, sem_b)
    finish(src_a, dst_a, rows_a, sem_a)
    start(2 * h + 2, src_a, dst_a, rows_a, sem_a)
    finish(src_b, dst_b, rows_b, sem_b)

  finish(src_a, dst_a, rows_a, sem_a)

  @pl.when(wid < _XTRA)
  def _():
    off = (_NW * _CPW + wid) * _CHUNK
    pltpu.sync_copy(edges.at[pl.ds(off, _CHUNK)], src_a)
    pltpu.sync_copy(edges.at[pl.ds(_E + off, _CHUNK)], dst_a)
    pltpu.sync_copy(data.at[src_a], rows_a)
    pltpu.sync_copy(rows_a, acc.at[dst_a], add=True)


def _make_seg_sum(width: int):
  mesh = plsc.VectorSubcoreMesh(core_axis_name="c", subcore_axis_name="s")
  out_type = jax.ShapeDtypeStruct((_NC, _NPAD, width), jnp.float32)
  scratch = [
      pltpu.VMEM((_CHUNK,), jnp.int32),
      pltpu.VMEM((_CHUNK,), jnp.int32),
      pltpu.VMEM((_CHUNK, width), jnp.float32),
      pltpu.SemaphoreType.DMA,
      pltpu.VMEM((_CHUNK,), jnp.int32),
      pltpu.VMEM((_CHUNK,), jnp.int32),
      pltpu.VMEM((_CHUNK, width), jnp.float32),
      pltpu.SemaphoreType.DMA,
      pltpu.VMEM_SHARED((_NPAD, width), jnp.float32),
  ]

  def body(data, edges, zeros, out,
           src_a, dst_a, rows_a, sem_a, src_b, dst_b, rows_b, sem_b, acc):
    c = lax.axis_index("c")
    s = lax.axis_index("s")
    base = s * _RPS

    pltpu.sync_copy(zeros, rows_a)

    @pl.loop(0, _ZCH)
    def _(k):
      pltpu.sync_copy(rows_a, acc.at[pl.ds(base + k * _CHUNK, _CHUNK), :])

    plsc.subcore_barrier()

    wid = c * _NS + s
    bufs = ((src_a, dst_a, rows_a, sem_a), (src_b, dst_b, rows_b, sem_b))
    _worker_edge_loop(edges, data, acc, bufs, wid)

    plsc.subcore_barrier()

    @pl.loop(0, _ZCH)
    def _(k):
      r = base + k * _CHUNK
      pltpu.sync_copy(acc.at[pl.ds(r, _CHUNK), :], rows_a)
      pltpu.sync_copy(rows_a, out.at[c, pl.ds(r, _CHUNK), :])

  return pl.kernel(body, out_type=out_type, mesh=mesh, scratch_types=scratch)


_seg_sum = functools.lru_cache(maxsize=None)(_make_seg_sum)


def _sibdiv_call(ssum, cnt):
  def body(ss_ref, cnt_ref, o_ref):
    total = ss_ref[0] + ss_ref[1]
    c = cnt_ref[0, :, 0:1] + cnt_ref[1, :, 0:1]
    o_ref[...] = total / jnp.maximum(c, 1.0)

  return pl.pallas_call(
      body,
      grid=(_GRID,),
      in_specs=[pl.BlockSpec((_NC, _BLK, _D), lambda i: (0, i, 0)),
                pl.BlockSpec((_NC, _BLK, _D), lambda i: (0, i, 0))],
      out_specs=pl.BlockSpec((_BLK, _D), lambda i: (i, 0)),
      out_shape=jax.ShapeDtypeStruct((_N, _D), jnp.float32),
  )(ssum, cnt)


def _level1_call(sib_h, lev2d, b2d):
  def body(sib_ref, lev_ref, b_ref, o_ref):
    upd = jnp.maximum(b_ref[...], 0.0) + sib_ref[...]
    o_ref[...] = jnp.where(lev_ref[...] == 1, upd, 0.0)

  return pl.pallas_call(
      body,
      grid=(_GRID,),
      in_specs=[pl.BlockSpec((_BLK, _D), lambda i: (i, 0)),
                pl.BlockSpec((_BLK, 1), lambda i: (i, 0)),
                pl.BlockSpec((1, _D), lambda i: (0, 0))],
      out_specs=pl.BlockSpec((_BLK, _D), lambda i: (i, 0)),
      out_shape=jax.ShapeDtypeStruct((_N, _D), jnp.float32),
  )(sib_h, lev2d, b2d)


def _update_call(lvl, z, pz, sib_h, lev2d, W, b2d):
  def body(z_ref, pz_ref, sib_ref, lev_ref, w_ref, b_ref, o_ref):
    acc = pz_ref[0] + pz_ref[1]
    h = jnp.dot(acc, w_ref[...], preferred_element_type=jnp.float32) + b_ref[...]
    upd = jnp.maximum(h, 0.0) + sib_ref[...]
    o_ref[...] = z_ref[...] + jnp.where(lev_ref[...] == lvl, upd, 0.0)

  return pl.pallas_call(
      body,
      grid=(_GRID,),
      in_specs=[pl.BlockSpec((_BLK, _D), lambda i: (i, 0)),
                pl.BlockSpec((_NC, _BLK, _D), lambda i: (0, i, 0)),
                pl.BlockSpec((_BLK, _D), lambda i: (i, 0)),
                pl.BlockSpec((_BLK, 1), lambda i: (i, 0)),
                pl.BlockSpec((_D, _D), lambda i: (0, 0)),
                pl.BlockSpec((1, _D), lambda i: (0, 0))],
      out_specs=pl.BlockSpec((_BLK, _D), lambda i: (i, 0)),
      out_shape=jax.ShapeDtypeStruct((_N, _D), jnp.float32),
  )(z, pz, sib_h, lev2d, W, b2d)


def _logits_call(z, Wp, bp, cpad):
  def body(z_ref, w_ref, b_ref, o_ref):
    o_ref[...] = jnp.dot(z_ref[...], w_ref[...],
                         preferred_element_type=jnp.float32) + b_ref[...]

  return pl.pallas_call(
      body,
      grid=(_GRID,),
      in_specs=[pl.BlockSpec((_BLK, _D), lambda i: (i, 0)),
                pl.BlockSpec((_D, cpad), lambda i: (0, 0)),
                pl.BlockSpec((1, cpad), lambda i: (0, 0))],
      out_specs=pl.BlockSpec((_BLK, cpad), lambda i: (i, 0)),
      out_shape=jax.ShapeDtypeStruct((_N, cpad), jnp.float32),
  )(z, Wp, bp)


def kernel(mol_vec, parent_edge_index, sibling_edge_index, node_level,
           W_parent, b_parent, W_cls, b_cls):
  zeros128 = jnp.zeros((_CHUNK, _D), jnp.float32)
  ones_tab = jnp.ones((_N, _D), jnp.float32)

  s_edges = sibling_edge_index.astype(jnp.int32).reshape(2 * _E)
  seg = _seg_sum(_D)
  sib_sum = seg(mol_vec, s_edges, zeros128)
  sib_cnt = seg(ones_tab, s_edges, zeros128)
  sib_h = _sibdiv_call(sib_sum, sib_cnt)

  lev2d = node_level.reshape(_N, 1).astype(jnp.int32)
  b2d = b_parent.reshape(1, _D)

  z = _level1_call(sib_h, lev2d, b2d)

  p_edges = parent_edge_index.astype(jnp.int32).reshape(2 * _E)
  for lvl in range(2, _NUM_LEVELS):
    pz = seg(z, p_edges, zeros128)
    z = _update_call(lvl, z, pz, sib_h, lev2d, W_parent, b2d)

  ncls = W_cls.shape[1]
  cpad = -(-ncls // 128) * 128
  Wp = jnp.pad(W_cls, ((0, 0), (0, cpad - ncls)))
  bp = jnp.pad(b_cls, (0, cpad - ncls)).reshape(1, cpad)
  logits = _logits_call(z, Wp, bp, cpad)
  return logits[:, :ncls]
